# baseline (speedup 1.0000x reference)
import jax
import jax.numpy as jnp
from jax.experimental import pallas as pl
from jax.experimental.pallas import tpu as pltpu

_CHUNK = 2000
_DEPTH = 4
_WAYS = 2
_HALF = _CHUNK // _WAYS


def _fused_linear_relu(h_hbm, w_ref, b_ref, o_hbm, ibuf, obuf, isem, osem):
    n = h_hbm.shape[0]
    nchunks = n // _CHUNK

    def in_copy(chunk, slot, way):
        return pltpu.make_async_copy(
            h_hbm.at[pl.ds(chunk * _CHUNK + way * _HALF, _HALF), :],
            ibuf.at[slot, pl.ds(way * _HALF, _HALF), :],
            isem.at[slot, way])

    def out_copy(chunk, slot, way):
        return pltpu.make_async_copy(
            obuf.at[slot, pl.ds(way * _HALF, _HALF), :],
            o_hbm.at[pl.ds(chunk * _CHUNK + way * _HALF, _HALF), :],
            osem.at[slot, way])

    for s in range(min(_DEPTH, nchunks)):
        for w_ in range(_WAYS):
            in_copy(s, s, w_).start()

    w = w_ref[...].astype(jnp.bfloat16)
    bias = b_ref[...]

    def body(i, carry):
        slot = jax.lax.rem(i, _DEPTH)
        for w_ in range(_WAYS):
            in_copy(i, slot, w_).wait()
        x = ibuf[slot].astype(jnp.bfloat16)
        y = jax.lax.dot_general(
            x, w, (((1,), (1,)), ((), ())),
            preferred_element_type=jnp.float32)
        y = jnp.maximum(y + bias, 0.0)

        @pl.when(i >= _DEPTH)
        def _wait_out():
            for w_ in range(_WAYS):
                out_copy(i - _DEPTH, slot, w_).wait()

        obuf[slot] = y
        for w_ in range(_WAYS):
            out_copy(i, slot, w_).start()

        @pl.when(i + _DEPTH < nchunks)
        def _next_in():
            for w_ in range(_WAYS):
                in_copy(i + _DEPTH, slot, w_).start()

        return carry

    jax.lax.fori_loop(0, nchunks, body, 0)

    for s in range(min(_DEPTH, nchunks)):
        i = nchunks - min(_DEPTH, nchunks) + s
        slot = i % _DEPTH
        for w_ in range(_WAYS):
            out_copy(i, slot, w_).wait()


def kernel(h, edge_index, W, b):
    del edge_index
    n, d_in = h.shape
    d_out = W.shape[0]
    b2 = b.reshape(1, d_out)
    return pl.pallas_call(
        _fused_linear_relu,
        in_specs=[
            pl.BlockSpec(memory_space=pl.ANY),
            pl.BlockSpec(memory_space=pltpu.MemorySpace.VMEM),
            pl.BlockSpec(memory_space=pltpu.MemorySpace.VMEM),
        ],
        out_specs=pl.BlockSpec(memory_space=pl.ANY),
        out_shape=jax.ShapeDtypeStruct((n, d_out), jnp.float32),
        scratch_shapes=[
            pltpu.VMEM((_DEPTH, _CHUNK, d_in), jnp.float32),
            pltpu.VMEM((_DEPTH, _CHUNK, d_out), jnp.float32),
            pltpu.SemaphoreType.DMA((_DEPTH, _WAYS)),
            pltpu.SemaphoreType.DMA((_DEPTH, _WAYS)),
        ],
    )(h, W, b2)


# speedup vs baseline: 1.0121x; 1.0121x over previous
import jax
import jax.numpy as jnp
from jax.experimental import pallas as pl
from jax.experimental.pallas import tpu as pltpu

_CHUNK = 5000
_DEPTH = 2
_WAYS = 1
_HALF = _CHUNK // _WAYS


def _fused_linear_relu(h_hbm, w_ref, b_ref, o_hbm, ibuf, obuf, isem, osem):
    n = h_hbm.shape[0]
    nchunks = n // _CHUNK

    def in_copy(chunk, slot, way):
        return pltpu.make_async_copy(
            h_hbm.at[pl.ds(chunk * _CHUNK + way * _HALF, _HALF), :],
            ibuf.at[slot, pl.ds(way * _HALF, _HALF), :],
            isem.at[slot, way])

    def out_copy(chunk, slot, way):
        return pltpu.make_async_copy(
            obuf.at[slot, pl.ds(way * _HALF, _HALF), :],
            o_hbm.at[pl.ds(chunk * _CHUNK + way * _HALF, _HALF), :],
            osem.at[slot, way])

    for s in range(min(_DEPTH, nchunks)):
        for w_ in range(_WAYS):
            in_copy(s, s, w_).start()

    w = w_ref[...].astype(jnp.bfloat16)
    bias = b_ref[...]

    def body(i, carry):
        slot = jax.lax.rem(i, _DEPTH)
        for w_ in range(_WAYS):
            in_copy(i, slot, w_).wait()
        x = ibuf[slot].astype(jnp.bfloat16)
        y = jax.lax.dot_general(
            x, w, (((1,), (1,)), ((), ())),
            preferred_element_type=jnp.float32)
        y = jnp.maximum(y + bias, 0.0)

        @pl.when(i >= _DEPTH)
        def _wait_out():
            for w_ in range(_WAYS):
                out_copy(i - _DEPTH, slot, w_).wait()

        obuf[slot] = y
        for w_ in range(_WAYS):
            out_copy(i, slot, w_).start()

        @pl.when(i + _DEPTH < nchunks)
        def _next_in():
            for w_ in range(_WAYS):
                in_copy(i + _DEPTH, slot, w_).start()

        return carry

    jax.lax.fori_loop(0, nchunks, body, 0)

    for s in range(min(_DEPTH, nchunks)):
        i = nchunks - min(_DEPTH, nchunks) + s
        slot = i % _DEPTH
        for w_ in range(_WAYS):
            out_copy(i, slot, w_).wait()


def kernel(h, edge_index, W, b):
    del edge_index
    n, d_in = h.shape
    d_out = W.shape[0]
    b2 = b.reshape(1, d_out)
    return pl.pallas_call(
        _fused_linear_relu,
        in_specs=[
            pl.BlockSpec(memory_space=pl.ANY),
            pl.BlockSpec(memory_space=pltpu.MemorySpace.VMEM),
            pl.BlockSpec(memory_space=pltpu.MemorySpace.VMEM),
        ],
        out_specs=pl.BlockSpec(memory_space=pl.ANY),
        out_shape=jax.ShapeDtypeStruct((n, d_out), jnp.float32),
        scratch_shapes=[
            pltpu.VMEM((_DEPTH, _CHUNK, d_in), jnp.float32),
            pltpu.VMEM((_DEPTH, _CHUNK, d_out), jnp.float32),
            pltpu.SemaphoreType.DMA((_DEPTH, _WAYS)),
            pltpu.SemaphoreType.DMA((_DEPTH, _WAYS)),
        ],
    )(h, W, b2)
